# baseline (speedup 1.0000x reference)
import functools

import jax
import jax.numpy as jnp
from jax import lax
from jax.experimental import pallas as pl
from jax.experimental.pallas import tpu as pltpu
from jax.experimental.pallas import tpu_sc as plsc

_NUM_ROWS = 1025
_DIM = 768


def kernel(table):
    info = plsc.get_sparse_core_info()
    nc, ns = info.num_cores, info.num_subcores
    nw = nc * ns
    rows_per_w = _NUM_ROWS // nw
    chunk = rows_per_w // 4
    rem = _NUM_ROWS - rows_per_w * nw
    tail = nw * rows_per_w

    mesh = plsc.VectorSubcoreMesh(core_axis_name="c", subcore_axis_name="s")

    @functools.partial(
        pl.kernel,
        mesh=mesh,
        out_type=jax.ShapeDtypeStruct((1, _NUM_ROWS, _DIM), jnp.float32),
        scratch_types=[
            pltpu.VMEM((chunk, _DIM), jnp.float32),
            pltpu.VMEM((chunk, _DIM), jnp.float32),
            pltpu.VMEM((rem, _DIM), jnp.float32),
            pltpu.SemaphoreType.DMA,
            pltpu.SemaphoreType.DMA,
            pltpu.SemaphoreType.DMA,
        ],
    )
    def copy_rows(table_hbm, out_hbm, buf0, buf1, tail_buf, sem0, sem1, sem2):
        wid = lax.axis_index("s") * nc + lax.axis_index("c")
        base = wid * rows_per_w
        bufs = (buf0, buf1)
        sems = (sem0, sem1)
        quarter = chunk
        n_chunks = rows_per_w // quarter

        gathers = [None, None]
        scatters = [None, None]
        for i in range(n_chunks):
            k = i % 2
            if scatters[k] is not None:
                scatters[k].wait()
            gathers[k] = pltpu.async_copy(
                table_hbm.at[pl.ds(base + i * quarter, quarter)], bufs[k], sems[k]
            )
            if i >= 1:
                j = (i - 1) % 2
                gathers[j].wait()
                scatters[j] = pltpu.async_copy(
                    bufs[j],
                    out_hbm.at[0, pl.ds(base + (i - 1) * quarter, quarter)],
                    sems[j],
                )
        last = n_chunks - 1
        gathers[last % 2].wait()
        scatters[last % 2] = pltpu.async_copy(
            bufs[last % 2],
            out_hbm.at[0, pl.ds(base + last * quarter, quarter)],
            sems[last % 2],
        )

        @pl.when(wid == nw - 1)
        def _copy_tail():
            gt = pltpu.async_copy(table_hbm.at[pl.ds(tail, rem)], tail_buf, sem2)
            gt.wait()
            pltpu.async_copy(tail_buf, out_hbm.at[0, pl.ds(tail, rem)], sem2).wait()

        scatters[0].wait()
        scatters[1].wait()

    return copy_rows(table)
